# baseline (speedup 1.0000x reference)
import functools

import jax
import jax.numpy as jnp
from jax.experimental import pallas as pl
from jax.experimental.pallas import tpu as pltpu
from jax.experimental.pallas import tpu_sc as plsc

DIM = 2048
INTER = 2048
E = 8
TOPK = 2
RANK = 128
B = 2
T = 2048
M = B * T
QMIN, QMAX = -128.0, 127.0
BALANCE_W, ENTROPY_W, ZLOSS_W = 0.3, 0.1, 0.0001

def _dot_t(a, b):
    return jax.lax.dot_general(a.astype(jnp.bfloat16), b.astype(jnp.bfloat16),
                               (((1,), (1,)), ((), ())),
                               preferred_element_type=jnp.float32)


def _fqa(v, s, z):
    q = jnp.clip(jnp.round(v / s + z), QMIN, QMAX)
    return (q - z) * s


def _mean_kernel(x_ref, o_ref):
    @pl.when(pl.program_id(0) == 0)
    def _():
        o_ref[...] = jnp.zeros_like(o_ref)

    o_ref[...] += jnp.sum(x_ref[...], axis=1) * (1.0 / T)


def _mean(x):
    bt = 256
    return pl.pallas_call(
        _mean_kernel,
        grid=(T // bt,),
        in_specs=[pl.BlockSpec((B, bt, DIM), lambda t: (0, t, 0))],
        out_specs=pl.BlockSpec((B, DIM), lambda t: (0, 0)),
        out_shape=jax.ShapeDtypeStruct((B, DIM), jnp.float32),
    )(x)


def _router_kernel(xm_ref, wp1_ref, bp1_ref, wp2_ref, bp2_ref, ws_ref, bs_ref,
                   temp_ref, s_out, l_out):
    xm = xm_ref[...]
    h = jnp.maximum(_dot_t(xm, wp1_ref[...]) + bp1_ref[...], 0.0)
    pol_l = _dot_t(h, wp2_ref[...]) + bp2_ref[...]
    pol_m = jnp.max(pol_l, axis=-1, keepdims=True)
    pol_e = jnp.exp(pol_l - pol_m)
    policy = pol_e / jnp.sum(pol_e, axis=-1, keepdims=True)

    temp = jnp.maximum(temp_ref[0], 0.1)
    base = (_dot_t(xm, ws_ref[...]) + bs_ref[...]) / temp

    sc_l = (base + policy) * 0.5
    sc_m = jnp.max(sc_l, axis=-1, keepdims=True)
    sc_e = jnp.exp(sc_l - sc_m)
    scores = sc_e / jnp.sum(sc_e, axis=-1, keepdims=True)

    entropy = -jnp.sum(scores * jnp.log(scores + 1e-6), axis=-1, keepdims=True)
    entropy_loss = -ENTROPY_W * jnp.mean(entropy)

    b_m = jnp.max(base, axis=-1, keepdims=True)
    lse = jnp.log(jnp.sum(jnp.exp(base - b_m), axis=-1, keepdims=True)) + b_m
    z_loss = ZLOSS_W * jnp.mean(lse * lse)

    s_out[...] = scores
    l_out[...] = jnp.full((1, 1), entropy_loss + z_loss, jnp.float32)


def _router(xm, rp):
    return pl.pallas_call(
        _router_kernel,
        in_specs=[pl.BlockSpec(memory_space=pltpu.VMEM)] * 7
        + [pl.BlockSpec(memory_space=pltpu.SMEM)],
        out_specs=[pl.BlockSpec(memory_space=pltpu.VMEM)] * 2,
        out_shape=[
            jax.ShapeDtypeStruct((B, E), jnp.float32),
            jax.ShapeDtypeStruct((1, 1), jnp.float32),
        ],
    )(xm, rp['Wp1'], rp['bp1'].reshape(1, 256), rp['Wp2'],
      rp['bp2'].reshape(1, E), rp['Ws'], rp['bs'].reshape(1, E),
      rp['temp'].reshape(1))


def _sc_route(scores16, aux16, temp16):
    @functools.partial(
        pl.kernel,
        out_type=[
            jax.ShapeDtypeStruct((B, 16), jnp.float32),
            jax.ShapeDtypeStruct((B, 16), jnp.int32),
            jax.ShapeDtypeStruct((16,), jnp.float32),
        ],
        mesh=plsc.VectorSubcoreMesh(core_axis_name="c", subcore_axis_name="s"),
        compiler_params=pltpu.CompilerParams(needs_layout_passes=False),
        scratch_types=[
            pltpu.VMEM((16,), jnp.float32),
            pltpu.VMEM((16,), jnp.float32),
            pltpu.VMEM((16,), jnp.int32),
            pltpu.VMEM((16,), jnp.float32),
            pltpu.VMEM((16,), jnp.float32),
            pltpu.VMEM((16,), jnp.float32),
        ],
    )
    def k(scores_hbm, aux_hbm, temp_hbm, w_hbm, i_hbm, bal_hbm,
          row_v, w_v, i_v, usage_v, bal_v, temp_v):
        @pl.when((jax.lax.axis_index("c") == 0)
                 & (jax.lax.axis_index("s") == 0))
        def _():
            usage_v[...] = jnp.zeros((16,), jnp.float32)
            pltpu.sync_copy(temp_hbm, temp_v)
            lanes = jax.lax.iota(jnp.int32, 16)
            selmask = lanes < TOPK
            for b in range(B):
                pltpu.sync_copy(scores_hbm.at[b], row_v)
                row = row_v[...]
                w1 = jnp.max(row)
                i1 = jnp.min(jnp.where(row == w1, lanes, 16))
                row2 = jnp.where(lanes == i1, -1.0, row)
                w2 = jnp.max(row2)
                i2 = jnp.min(jnp.where(row2 == w2, lanes, 16))
                wv = jnp.where(lanes == 0, w1,
                               jnp.where(lanes == 1, w2, 0.0)) * temp_v[...]
                iv = jnp.where(lanes == 0, i1, jnp.where(lanes == 1, i2, 0))
                w_v[...] = wv
                i_v[...] = iv
                pltpu.sync_copy(w_v, w_hbm.at[b])
                pltpu.sync_copy(i_v, i_hbm.at[b])
                plsc.addupdate_scatter(usage_v, [iv], wv, mask=selmask)
            u = usage_v[...]
            zv = jnp.zeros((16,), jnp.float32)
            meanv = zv + jnp.sum(u) * (1.0 / E)
            frac = u / (meanv + 1e-6)
            emask = lanes < E
            fr = jnp.where(emask, frac, 0.0)
            muv = zv + jnp.sum(fr) * (1.0 / E)
            d = jnp.where(emask, fr - muv, 0.0)
            varv = zv + jnp.sum(d * d) * (1.0 / (E - 1))
            pltpu.sync_copy(aux_hbm, bal_v)
            bal_v[...] = BALANCE_W * varv + bal_v[...]
            pltpu.sync_copy(bal_v, bal_hbm)

    return k(scores16, aux16, temp16)


def _wminmax_kernel(ids_ref, w_ref, mn_ref, mx_ref):
    @pl.when(pl.program_id(1) == 0)
    def _():
        mn_ref[...] = jnp.zeros_like(mn_ref)
        mx_ref[...] = jnp.zeros_like(mx_ref)

    w = w_ref[0]
    mn_ref[...] = jnp.minimum(mn_ref[...], jnp.min(w))
    mx_ref[...] = jnp.maximum(mx_ref[...], jnp.max(w))


def _wminmax(wst, ids, nslots):
    out_f = wst.shape[1]
    br = 512
    grid_spec = pltpu.PrefetchScalarGridSpec(
        num_scalar_prefetch=1,
        grid=(nslots, out_f // br),
        in_specs=[pl.BlockSpec((1, br, wst.shape[2]),
                               lambda s, r, ids: (ids[s], r, 0))],
        out_specs=[pl.BlockSpec((1, 1, 1), lambda s, r, ids: (s, 0, 0)),
                   pl.BlockSpec((1, 1, 1), lambda s, r, ids: (s, 0, 0))],
    )
    mn, mx = pl.pallas_call(
        _wminmax_kernel,
        grid_spec=grid_spec,
        out_shape=[jax.ShapeDtypeStruct((nslots, 1, 1), jnp.float32)] * 2,
    )(ids, wst)
    return mn[:, 0, 0], mx[:, 0, 0]


def _wquant_kernel(ids_ref, sw_ref, zw_ref, w_ref, wq_ref):
    s = pl.program_id(0)
    sw = sw_ref[s]
    zw = zw_ref[s]
    q = jnp.clip(jnp.round(w_ref[0] / sw + zw), QMIN, QMAX) - zw
    wq_ref[0] = (q * sw).astype(jnp.bfloat16)


def _wquant(wst, ids, sw, zw, nslots):
    out_f, k = wst.shape[1], wst.shape[2]
    br = 512
    grid_spec = pltpu.PrefetchScalarGridSpec(
        num_scalar_prefetch=1,
        grid=(nslots, out_f // br),
        in_specs=[pl.BlockSpec(memory_space=pltpu.SMEM),
                  pl.BlockSpec(memory_space=pltpu.SMEM),
                  pl.BlockSpec((1, br, k), lambda s, r, ids: (ids[s], r, 0))],
        out_specs=pl.BlockSpec((1, br, k), lambda s, r, ids: (s, r, 0)),
    )
    return pl.pallas_call(
        _wquant_kernel,
        grid_spec=grid_spec,
        out_shape=jax.ShapeDtypeStruct((nslots, out_f, k), jnp.bfloat16),
    )(ids, sw, zw, wst)


def _main_kernel(ids_ref, x_ref, wq_ref, b_ref, a_ref, o_ref, mn_ref, mx_ref):
    n = pl.program_id(1)
    m = pl.program_id(2)

    x = x_ref[0]
    p = _dot_t(x, b_ref[0]).astype(jnp.bfloat16)
    acc = _dot_t(x, wq_ref[0])
    acc = acc + _dot_t(p, a_ref[0])
    o_ref[0] = acc

    @pl.when((n == 0) & (m == 0))
    def _():
        mn_ref[...] = jnp.zeros_like(mn_ref)
        mx_ref[...] = jnp.zeros_like(mx_ref)

    mn_ref[...] = jnp.minimum(mn_ref[...], jnp.min(acc))
    mx_ref[...] = jnp.maximum(mx_ref[...], jnp.max(acc))


def _main_mm(x3, wq, bst, ast, ids, nslots, x_per_slot):
    bm, bn = 1024, 2048
    out_f = wq.shape[1]
    k = wq.shape[2]
    if x_per_slot:
        x_imap = lambda s, n, m, ids: (s, m, 0)
    else:
        x_imap = lambda s, n, m, ids: (0, m, 0)
    grid_spec = pltpu.PrefetchScalarGridSpec(
        num_scalar_prefetch=1,
        grid=(nslots, out_f // bn, M // bm),
        in_specs=[
            pl.BlockSpec((1, bm, k), x_imap),
            pl.BlockSpec((1, bn, k), lambda s, n, m, ids: (s, n, 0)),
            pl.BlockSpec((1, RANK, k), lambda s, n, m, ids: (ids[s], 0, 0)),
            pl.BlockSpec((1, bn, RANK), lambda s, n, m, ids: (ids[s], n, 0)),
        ],
        out_specs=[
            pl.BlockSpec((1, bm, bn), lambda s, n, m, ids: (s, m, n)),
            pl.BlockSpec((1, 1, 1), lambda s, n, m, ids: (s, 0, 0)),
            pl.BlockSpec((1, 1, 1), lambda s, n, m, ids: (s, 0, 0)),
        ],
    )
    return pl.pallas_call(
        _main_kernel,
        grid_spec=grid_spec,
        out_shape=[
            jax.ShapeDtypeStruct((nslots, M, out_f), jnp.float32),
            jax.ShapeDtypeStruct((nslots, 1, 1), jnp.float32),
            jax.ShapeDtypeStruct((nslots, 1, 1), jnp.float32),
        ],
    )(ids, x3, wq, bst, ast)


def _h_kernel(s1_ref, z1_ref, s3_ref, z3_ref, r1_ref, r3_ref, h_ref):
    s = pl.program_id(0)
    d1 = _fqa(r1_ref[0], s1_ref[s], z1_ref[s])
    d3 = _fqa(r3_ref[0], s3_ref[s], z3_ref[s])
    g = 1.0 / (1.0 + jnp.exp(-d3))
    h_ref[0] = (d1 * g).astype(jnp.bfloat16)


def _h_stage(r1, r3, s1, z1, s3, z3, nslots):
    bm = 512
    return pl.pallas_call(
        _h_kernel,
        grid=(nslots, M // bm),
        in_specs=[pl.BlockSpec(memory_space=pltpu.SMEM)] * 4
        + [pl.BlockSpec((1, bm, INTER), lambda s, m: (s, m, 0)),
           pl.BlockSpec((1, bm, INTER), lambda s, m: (s, m, 0))],
        out_specs=pl.BlockSpec((1, bm, INTER), lambda s, m: (s, m, 0)),
        out_shape=jax.ShapeDtypeStruct((nslots, M, INTER), jnp.bfloat16),
    )(s1, z1, s3, z3, r1, r3)


def _combine_kernel(g_ref, sa_ref, za_ref, sb_ref, zb_ref,
                    r2e_ref, r2s_ref, y_ref, *, bm):
    m = pl.program_id(0)
    b = (m * bm) // T
    acc = _fqa(_fqa(r2s_ref[...], sa_ref[2 * TOPK], za_ref[2 * TOPK]),
               sb_ref[2 * TOPK], zb_ref[2 * TOPK])
    for s in range(B * TOPK):
        v = _fqa(_fqa(r2e_ref[s], sa_ref[s], za_ref[s]),
                 sb_ref[s], zb_ref[s])
        acc = acc + g_ref[s, b] * v
    y_ref[...] = acc


def _combine(r2e, r2s, gmat, sa, za, sb, zb):
    bm, bn = 512, 512
    return pl.pallas_call(
        functools.partial(_combine_kernel, bm=bm),
        grid=(M // bm, DIM // bn),
        in_specs=[pl.BlockSpec(memory_space=pltpu.SMEM)] * 5
        + [pl.BlockSpec((B * TOPK, bm, bn), lambda m, n: (0, m, n)),
           pl.BlockSpec((bm, bn), lambda m, n: (m, n))],
        out_specs=pl.BlockSpec((bm, bn), lambda m, n: (m, n)),
        out_shape=jax.ShapeDtypeStruct((M, DIM), jnp.float32),
    )(gmat, sa, za, sb, zb, r2e, r2s)


def _scale_zp(mn, mx):
    s = jnp.maximum((mx - mn) / (QMAX - QMIN), 1e-8)
    z = jnp.clip(jnp.round(QMIN - mn / s), QMIN, QMAX)
    return s, z


def _lora_layer(x3, wp, ids, nslots, x_per_slot):
    wst, ast, bst = wp['weight'], wp['lora_A'], wp['lora_B']
    if wst.ndim == 2:
        wst = wst.reshape((1,) + wst.shape)
        ast = ast.reshape((1,) + ast.shape)
        bst = bst.reshape((1,) + bst.shape)
    mnw, mxw = _wminmax(wst, ids, nslots)
    sw, zw = _scale_zp(mnw, mxw)
    wq = _wquant(wst, ids, sw, zw, nslots)
    r, mn, mx = _main_mm(x3, wq, bst, ast, ids, nslots, x_per_slot)
    return r, mn[:, 0, 0], mx[:, 0, 0]


def _expert_stack(x3, prm, ids, nslots, x_per_slot):
    r3, mn3, mx3 = _lora_layer(x3, prm['w3'], ids, nslots, x_per_slot)
    r1, mn1, mx1 = _lora_layer(x3, prm['w1'], ids, nslots, x_per_slot)
    s1, z1 = _scale_zp(mn1, mx1)
    s3, z3 = _scale_zp(mn3, mx3)
    h = _h_stage(r1, r3, s1, z1, s3, z3, nslots)
    r2, mn2, mx2 = _lora_layer(h, prm['w2'], ids, nslots, True)
    return r2, mn2, mx2


def kernel(x, params):
    X3 = x.reshape(1, M, DIM).astype(jnp.bfloat16)
    xm = _mean(x)
    scores, aux = _router(xm, params['router'])

    scores16 = jnp.concatenate(
        [scores, jnp.full((B, 16 - E), -1.0, jnp.float32)], axis=1)
    aux16 = jnp.full((16,), aux[0, 0], jnp.float32)
    temp16 = jnp.full((16,), jnp.maximum(params['router']['temp'], 0.1),
                      jnp.float32)
    w16, i16, bal16 = _sc_route(scores16, aux16, temp16)
    wts = w16[:, :TOPK]
    idx = i16[:, :TOPK]

    ids = idx.reshape(-1).astype(jnp.int32)
    ids0 = jnp.zeros((1,), jnp.int32)

    r2e, mn2e, mx2e = _expert_stack(X3, params['experts'], ids, B * TOPK, False)
    r2s, mn2s, mx2s = _expert_stack(X3, params['shared'], ids0, 1, False)

    mn2 = jnp.concatenate([mn2e, mn2s])
    mx2 = jnp.concatenate([mx2e, mx2s])
    sa, za = _scale_zp(mn2, mx2)
    dmn = _fqa(mn2, sa, za)
    dmx = _fqa(mx2, sa, za)
    sb, zb = _scale_zp(dmn, dmx)

    gvec = wts.reshape(-1)
    sb_of_slot = jnp.arange(B * TOPK) // TOPK
    gmat = gvec[:, None] * (sb_of_slot[:, None] ==
                            jnp.arange(B)[None, :]).astype(jnp.float32)

    y = _combine(r2e, r2s[0], gmat, sa, za, sb, zb)
    return y.reshape(B, T, DIM), bal16[0]


# speedup vs baseline: 1.0769x; 1.0769x over previous
import functools

import jax
import jax.numpy as jnp
from jax.experimental import pallas as pl
from jax.experimental.pallas import tpu as pltpu
from jax.experimental.pallas import tpu_sc as plsc

DIM = 2048
INTER = 2048
E = 8
TOPK = 2
RANK = 128
B = 2
T = 2048
M = B * T
QMIN, QMAX = -128.0, 127.0
BALANCE_W, ENTROPY_W, ZLOSS_W = 0.3, 0.1, 0.0001

def _dot_t(a, b):
    return jax.lax.dot_general(a.astype(jnp.bfloat16), b.astype(jnp.bfloat16),
                               (((1,), (1,)), ((), ())),
                               preferred_element_type=jnp.float32)


def _fqa(v, s, z):
    q = jnp.clip(jnp.round(v / s + z), QMIN, QMAX)
    return (q - z) * s


def _mean_kernel(x_ref, o_ref):
    @pl.when(pl.program_id(0) == 0)
    def _():
        o_ref[...] = jnp.zeros_like(o_ref)

    o_ref[...] += jnp.sum(x_ref[...], axis=1) * (1.0 / T)


def _mean(x):
    bt = 256
    return pl.pallas_call(
        _mean_kernel,
        grid=(T // bt,),
        in_specs=[pl.BlockSpec((B, bt, DIM), lambda t: (0, t, 0))],
        out_specs=pl.BlockSpec((B, DIM), lambda t: (0, 0)),
        out_shape=jax.ShapeDtypeStruct((B, DIM), jnp.float32),
    )(x)


def _router_kernel(xm_ref, wp1_ref, bp1_ref, wp2_ref, bp2_ref, ws_ref, bs_ref,
                   temp_ref, s_out, l_out):
    xm = xm_ref[...]
    h = jnp.maximum(_dot_t(xm, wp1_ref[...]) + bp1_ref[...], 0.0)
    pol_l = _dot_t(h, wp2_ref[...]) + bp2_ref[...]
    pol_m = jnp.max(pol_l, axis=-1, keepdims=True)
    pol_e = jnp.exp(pol_l - pol_m)
    policy = pol_e / jnp.sum(pol_e, axis=-1, keepdims=True)

    temp = jnp.maximum(temp_ref[0], 0.1)
    base = (_dot_t(xm, ws_ref[...]) + bs_ref[...]) / temp

    sc_l = (base + policy) * 0.5
    sc_m = jnp.max(sc_l, axis=-1, keepdims=True)
    sc_e = jnp.exp(sc_l - sc_m)
    scores = sc_e / jnp.sum(sc_e, axis=-1, keepdims=True)

    entropy = -jnp.sum(scores * jnp.log(scores + 1e-6), axis=-1, keepdims=True)
    entropy_loss = -ENTROPY_W * jnp.mean(entropy)

    b_m = jnp.max(base, axis=-1, keepdims=True)
    lse = jnp.log(jnp.sum(jnp.exp(base - b_m), axis=-1, keepdims=True)) + b_m
    z_loss = ZLOSS_W * jnp.mean(lse * lse)

    s_out[...] = scores
    l_out[...] = jnp.full((1, 1), entropy_loss + z_loss, jnp.float32)


def _router(xm, rp):
    return pl.pallas_call(
        _router_kernel,
        in_specs=[pl.BlockSpec(memory_space=pltpu.VMEM)] * 7
        + [pl.BlockSpec(memory_space=pltpu.SMEM)],
        out_specs=[pl.BlockSpec(memory_space=pltpu.VMEM)] * 2,
        out_shape=[
            jax.ShapeDtypeStruct((B, E), jnp.float32),
            jax.ShapeDtypeStruct((1, 1), jnp.float32),
        ],
    )(xm, rp['Wp1'], rp['bp1'].reshape(1, 256), rp['Wp2'],
      rp['bp2'].reshape(1, E), rp['Ws'], rp['bs'].reshape(1, E),
      rp['temp'].reshape(1))


def _sc_route(scores16, aux16, temp16):
    @functools.partial(
        pl.kernel,
        out_type=[
            jax.ShapeDtypeStruct((B, 16), jnp.float32),
            jax.ShapeDtypeStruct((B, 16), jnp.int32),
            jax.ShapeDtypeStruct((16,), jnp.float32),
        ],
        mesh=plsc.VectorSubcoreMesh(core_axis_name="c", subcore_axis_name="s"),
        compiler_params=pltpu.CompilerParams(needs_layout_passes=False),
        scratch_types=[
            pltpu.VMEM((16,), jnp.float32),
            pltpu.VMEM((16,), jnp.float32),
            pltpu.VMEM((16,), jnp.int32),
            pltpu.VMEM((16,), jnp.float32),
            pltpu.VMEM((16,), jnp.float32),
            pltpu.VMEM((16,), jnp.float32),
        ],
    )
    def k(scores_hbm, aux_hbm, temp_hbm, w_hbm, i_hbm, bal_hbm,
          row_v, w_v, i_v, usage_v, bal_v, temp_v):
        @pl.when((jax.lax.axis_index("c") == 0)
                 & (jax.lax.axis_index("s") == 0))
        def _():
            usage_v[...] = jnp.zeros((16,), jnp.float32)
            pltpu.sync_copy(temp_hbm, temp_v)
            lanes = jax.lax.iota(jnp.int32, 16)
            selmask = lanes < TOPK
            for b in range(B):
                pltpu.sync_copy(scores_hbm.at[b], row_v)
                row = row_v[...]
                w1 = jnp.max(row)
                i1 = jnp.min(jnp.where(row == w1, lanes, 16))
                row2 = jnp.where(lanes == i1, -1.0, row)
                w2 = jnp.max(row2)
                i2 = jnp.min(jnp.where(row2 == w2, lanes, 16))
                wv = jnp.where(lanes == 0, w1,
                               jnp.where(lanes == 1, w2, 0.0)) * temp_v[...]
                iv = jnp.where(lanes == 0, i1, jnp.where(lanes == 1, i2, 0))
                w_v[...] = wv
                i_v[...] = iv
                pltpu.sync_copy(w_v, w_hbm.at[b])
                pltpu.sync_copy(i_v, i_hbm.at[b])
                plsc.addupdate_scatter(usage_v, [iv], wv, mask=selmask)
            u = usage_v[...]
            zv = jnp.zeros((16,), jnp.float32)
            meanv = zv + jnp.sum(u) * (1.0 / E)
            frac = u / (meanv + 1e-6)
            emask = lanes < E
            fr = jnp.where(emask, frac, 0.0)
            muv = zv + jnp.sum(fr) * (1.0 / E)
            d = jnp.where(emask, fr - muv, 0.0)
            varv = zv + jnp.sum(d * d) * (1.0 / (E - 1))
            pltpu.sync_copy(aux_hbm, bal_v)
            bal_v[...] = BALANCE_W * varv + bal_v[...]
            pltpu.sync_copy(bal_v, bal_hbm)

    return k(scores16, aux16, temp16)


def _wprep_kernel(ids_ref, w_ref, wq_ref, *, rows, chunk):
    mn = jnp.float32(0.0)
    mx = jnp.float32(0.0)
    for c in range(rows // chunk):
        blk = w_ref[0, pl.ds(c * chunk, chunk), :]
        mn = jnp.minimum(mn, jnp.min(blk))
        mx = jnp.maximum(mx, jnp.max(blk))
    sw = jnp.maximum((mx - mn) / (QMAX - QMIN), 1e-8)
    zw = jnp.clip(jnp.round(QMIN - mn / sw), QMIN, QMAX)
    for c in range(rows // chunk):
        blk = w_ref[0, pl.ds(c * chunk, chunk), :]
        q = jnp.clip(jnp.round(blk / sw + zw), QMIN, QMAX) - zw
        wq_ref[0, pl.ds(c * chunk, chunk), :] = (q * sw).astype(jnp.bfloat16)


def _wprep(wst, ids, nslots):
    out_f, k = wst.shape[1], wst.shape[2]
    grid_spec = pltpu.PrefetchScalarGridSpec(
        num_scalar_prefetch=1,
        grid=(nslots,),
        in_specs=[pl.BlockSpec((1, out_f, k), lambda s, ids: (ids[s], 0, 0))],
        out_specs=pl.BlockSpec((1, out_f, k), lambda s, ids: (s, 0, 0)),
    )
    return pl.pallas_call(
        functools.partial(_wprep_kernel, rows=out_f, chunk=256),
        grid_spec=grid_spec,
        out_shape=jax.ShapeDtypeStruct((nslots, out_f, k), jnp.bfloat16),
    )(ids, wst)


def _main_kernel(ids_ref, x_ref, wq_ref, b_ref, a_ref, o_ref, mn_ref, mx_ref):
    n = pl.program_id(1)
    m = pl.program_id(2)

    x = x_ref[0]
    p = _dot_t(x, b_ref[0]).astype(jnp.bfloat16)
    acc = _dot_t(x, wq_ref[0])
    acc = acc + _dot_t(p, a_ref[0])
    o_ref[0] = acc

    @pl.when((n == 0) & (m == 0))
    def _():
        mn_ref[...] = jnp.zeros_like(mn_ref)
        mx_ref[...] = jnp.zeros_like(mx_ref)

    mn_ref[...] = jnp.minimum(mn_ref[...], jnp.min(acc))
    mx_ref[...] = jnp.maximum(mx_ref[...], jnp.max(acc))


def _main_mm(x3, wq, bst, ast, ids, nslots, x_per_slot):
    bm, bn = 1024, 2048
    out_f = wq.shape[1]
    k = wq.shape[2]
    if x_per_slot:
        x_imap = lambda s, n, m, ids: (s, m, 0)
    else:
        x_imap = lambda s, n, m, ids: (0, m, 0)
    grid_spec = pltpu.PrefetchScalarGridSpec(
        num_scalar_prefetch=1,
        grid=(nslots, out_f // bn, M // bm),
        in_specs=[
            pl.BlockSpec((1, bm, k), x_imap),
            pl.BlockSpec((1, bn, k), lambda s, n, m, ids: (s, n, 0)),
            pl.BlockSpec((1, RANK, k), lambda s, n, m, ids: (ids[s], 0, 0)),
            pl.BlockSpec((1, bn, RANK), lambda s, n, m, ids: (ids[s], n, 0)),
        ],
        out_specs=[
            pl.BlockSpec((1, bm, bn), lambda s, n, m, ids: (s, m, n)),
            pl.BlockSpec((1, 1, 1), lambda s, n, m, ids: (s, 0, 0)),
            pl.BlockSpec((1, 1, 1), lambda s, n, m, ids: (s, 0, 0)),
        ],
    )
    return pl.pallas_call(
        _main_kernel,
        grid_spec=grid_spec,
        out_shape=[
            jax.ShapeDtypeStruct((nslots, M, out_f), jnp.float32),
            jax.ShapeDtypeStruct((nslots, 1, 1), jnp.float32),
            jax.ShapeDtypeStruct((nslots, 1, 1), jnp.float32),
        ],
    )(ids, x3, wq, bst, ast)


def _h_kernel(s1_ref, z1_ref, s3_ref, z3_ref, r1_ref, r3_ref, h_ref):
    s = pl.program_id(0)
    d1 = _fqa(r1_ref[0], s1_ref[s], z1_ref[s])
    d3 = _fqa(r3_ref[0], s3_ref[s], z3_ref[s])
    g = 1.0 / (1.0 + jnp.exp(-d3))
    h_ref[0] = (d1 * g).astype(jnp.bfloat16)


def _h_stage(r1, r3, s1, z1, s3, z3, nslots):
    bm = 512
    return pl.pallas_call(
        _h_kernel,
        grid=(nslots, M // bm),
        in_specs=[pl.BlockSpec(memory_space=pltpu.SMEM)] * 4
        + [pl.BlockSpec((1, bm, INTER), lambda s, m: (s, m, 0)),
           pl.BlockSpec((1, bm, INTER), lambda s, m: (s, m, 0))],
        out_specs=pl.BlockSpec((1, bm, INTER), lambda s, m: (s, m, 0)),
        out_shape=jax.ShapeDtypeStruct((nslots, M, INTER), jnp.bfloat16),
    )(s1, z1, s3, z3, r1, r3)


def _combine_kernel(g_ref, sa_ref, za_ref, sb_ref, zb_ref,
                    r2e_ref, r2s_ref, y_ref, *, bm):
    m = pl.program_id(0)
    b = (m * bm) // T
    acc = _fqa(_fqa(r2s_ref[...], sa_ref[2 * TOPK], za_ref[2 * TOPK]),
               sb_ref[2 * TOPK], zb_ref[2 * TOPK])
    for s in range(B * TOPK):
        v = _fqa(_fqa(r2e_ref[s], sa_ref[s], za_ref[s]),
                 sb_ref[s], zb_ref[s])
        acc = acc + g_ref[s, b] * v
    y_ref[...] = acc


def _combine(r2e, r2s, gmat, sa, za, sb, zb):
    bm, bn = 512, 512
    return pl.pallas_call(
        functools.partial(_combine_kernel, bm=bm),
        grid=(M // bm, DIM // bn),
        in_specs=[pl.BlockSpec(memory_space=pltpu.SMEM)] * 5
        + [pl.BlockSpec((B * TOPK, bm, bn), lambda m, n: (0, m, n)),
           pl.BlockSpec((bm, bn), lambda m, n: (m, n))],
        out_specs=pl.BlockSpec((bm, bn), lambda m, n: (m, n)),
        out_shape=jax.ShapeDtypeStruct((M, DIM), jnp.float32),
    )(gmat, sa, za, sb, zb, r2e, r2s)


def _scale_zp(mn, mx):
    s = jnp.maximum((mx - mn) / (QMAX - QMIN), 1e-8)
    z = jnp.clip(jnp.round(QMIN - mn / s), QMIN, QMAX)
    return s, z


def _lora_layer(x3, wp, ids, nslots, x_per_slot):
    wst, ast, bst = wp['weight'], wp['lora_A'], wp['lora_B']
    if wst.ndim == 2:
        wst = wst.reshape((1,) + wst.shape)
        ast = ast.reshape((1,) + ast.shape)
        bst = bst.reshape((1,) + bst.shape)
    wq = _wprep(wst, ids, nslots)
    r, mn, mx = _main_mm(x3, wq, bst, ast, ids, nslots, x_per_slot)
    return r, mn[:, 0, 0], mx[:, 0, 0]


def _expert_stack(x3, prm, ids, nslots, x_per_slot):
    r3, mn3, mx3 = _lora_layer(x3, prm['w3'], ids, nslots, x_per_slot)
    r1, mn1, mx1 = _lora_layer(x3, prm['w1'], ids, nslots, x_per_slot)
    s1, z1 = _scale_zp(mn1, mx1)
    s3, z3 = _scale_zp(mn3, mx3)
    h = _h_stage(r1, r3, s1, z1, s3, z3, nslots)
    r2, mn2, mx2 = _lora_layer(h, prm['w2'], ids, nslots, True)
    return r2, mn2, mx2


def kernel(x, params):
    X3 = x.reshape(1, M, DIM).astype(jnp.bfloat16)
    xm = _mean(x)
    scores, aux = _router(xm, params['router'])

    scores16 = jnp.concatenate(
        [scores, jnp.full((B, 16 - E), -1.0, jnp.float32)], axis=1)
    aux16 = jnp.full((16,), aux[0, 0], jnp.float32)
    temp16 = jnp.full((16,), jnp.maximum(params['router']['temp'], 0.1),
                      jnp.float32)
    w16, i16, bal16 = _sc_route(scores16, aux16, temp16)
    wts = w16[:, :TOPK]
    idx = i16[:, :TOPK]

    ids = idx.reshape(-1).astype(jnp.int32)
    ids0 = jnp.zeros((1,), jnp.int32)

    r2s, mn2s, mx2s = _expert_stack(X3, params['shared'], ids0, 1, False)
    r2e, mn2e, mx2e = _expert_stack(X3, params['experts'], ids, B * TOPK, False)

    mn2 = jnp.concatenate([mn2e, mn2s])
    mx2 = jnp.concatenate([mx2e, mx2s])
    sa, za = _scale_zp(mn2, mx2)
    dmn = _fqa(mn2, sa, za)
    dmx = _fqa(mx2, sa, za)
    sb, zb = _scale_zp(dmn, dmx)

    gvec = wts.reshape(-1)
    sb_of_slot = jnp.arange(B * TOPK) // TOPK
    gmat = gvec[:, None] * (sb_of_slot[:, None] ==
                            jnp.arange(B)[None, :]).astype(jnp.float32)

    y = _combine(r2e, r2s[0], gmat, sa, za, sb, zb)
    return y.reshape(B, T, DIM), bal16[0]


# speedup vs baseline: 1.1457x; 1.0639x over previous
import functools

import jax
import jax.numpy as jnp
from jax.experimental import pallas as pl
from jax.experimental.pallas import tpu as pltpu
from jax.experimental.pallas import tpu_sc as plsc

DIM = 2048
INTER = 2048
E = 8
TOPK = 2
RANK = 128
B = 2
T = 2048
M = B * T
QMIN, QMAX = -128.0, 127.0
BALANCE_W, ENTROPY_W, ZLOSS_W = 0.3, 0.1, 0.0001

def _dot_t(a, b):
    return jax.lax.dot_general(a.astype(jnp.bfloat16), b.astype(jnp.bfloat16),
                               (((1,), (1,)), ((), ())),
                               preferred_element_type=jnp.float32)


def _fqa(v, s, z):
    q = jnp.clip(jnp.round(v / s + z), QMIN, QMAX)
    return (q - z) * s


def _mean_kernel(x_ref, o_ref):
    @pl.when(pl.program_id(0) == 0)
    def _():
        o_ref[...] = jnp.zeros_like(o_ref)

    o_ref[...] += jnp.sum(x_ref[...], axis=1) * (1.0 / T)


def _mean(x):
    bt = 256
    return pl.pallas_call(
        _mean_kernel,
        grid=(T // bt,),
        in_specs=[pl.BlockSpec((B, bt, DIM), lambda t: (0, t, 0))],
        out_specs=pl.BlockSpec((B, DIM), lambda t: (0, 0)),
        out_shape=jax.ShapeDtypeStruct((B, DIM), jnp.float32),
    )(x)


def _router_kernel(xm_ref, wp1_ref, bp1_ref, wp2_ref, bp2_ref, ws_ref, bs_ref,
                   temp_ref, s_out, l_out):
    xm = xm_ref[...]
    h = jnp.maximum(_dot_t(xm, wp1_ref[...]) + bp1_ref[...], 0.0)
    pol_l = _dot_t(h, wp2_ref[...]) + bp2_ref[...]
    pol_m = jnp.max(pol_l, axis=-1, keepdims=True)
    pol_e = jnp.exp(pol_l - pol_m)
    policy = pol_e / jnp.sum(pol_e, axis=-1, keepdims=True)

    temp = jnp.maximum(temp_ref[0], 0.1)
    base = (_dot_t(xm, ws_ref[...]) + bs_ref[...]) / temp

    sc_l = (base + policy) * 0.5
    sc_m = jnp.max(sc_l, axis=-1, keepdims=True)
    sc_e = jnp.exp(sc_l - sc_m)
    scores = sc_e / jnp.sum(sc_e, axis=-1, keepdims=True)

    entropy = -jnp.sum(scores * jnp.log(scores + 1e-6), axis=-1, keepdims=True)
    entropy_loss = -ENTROPY_W * jnp.mean(entropy)

    b_m = jnp.max(base, axis=-1, keepdims=True)
    lse = jnp.log(jnp.sum(jnp.exp(base - b_m), axis=-1, keepdims=True)) + b_m
    z_loss = ZLOSS_W * jnp.mean(lse * lse)

    s_out[...] = scores
    l_out[...] = jnp.full((1, 1), entropy_loss + z_loss, jnp.float32)


def _router(xm, rp):
    return pl.pallas_call(
        _router_kernel,
        in_specs=[pl.BlockSpec(memory_space=pltpu.VMEM)] * 7
        + [pl.BlockSpec(memory_space=pltpu.SMEM)],
        out_specs=[pl.BlockSpec(memory_space=pltpu.VMEM)] * 2,
        out_shape=[
            jax.ShapeDtypeStruct((B, E), jnp.float32),
            jax.ShapeDtypeStruct((1, 1), jnp.float32),
        ],
    )(xm, rp['Wp1'], rp['bp1'].reshape(1, 256), rp['Wp2'],
      rp['bp2'].reshape(1, E), rp['Ws'], rp['bs'].reshape(1, E),
      rp['temp'].reshape(1))


def _sc_route(scores16, aux16, temp16):
    @functools.partial(
        pl.kernel,
        out_type=[
            jax.ShapeDtypeStruct((B, 16), jnp.float32),
            jax.ShapeDtypeStruct((B, 16), jnp.int32),
            jax.ShapeDtypeStruct((16,), jnp.float32),
        ],
        mesh=plsc.VectorSubcoreMesh(core_axis_name="c", subcore_axis_name="s"),
        compiler_params=pltpu.CompilerParams(needs_layout_passes=False),
        scratch_types=[
            pltpu.VMEM((16,), jnp.float32),
            pltpu.VMEM((16,), jnp.float32),
            pltpu.VMEM((16,), jnp.int32),
            pltpu.VMEM((16,), jnp.float32),
            pltpu.VMEM((16,), jnp.float32),
            pltpu.VMEM((16,), jnp.float32),
        ],
    )
    def k(scores_hbm, aux_hbm, temp_hbm, w_hbm, i_hbm, bal_hbm,
          row_v, w_v, i_v, usage_v, bal_v, temp_v):
        @pl.when((jax.lax.axis_index("c") == 0)
                 & (jax.lax.axis_index("s") == 0))
        def _():
            usage_v[...] = jnp.zeros((16,), jnp.float32)
            pltpu.sync_copy(temp_hbm, temp_v)
            lanes = jax.lax.iota(jnp.int32, 16)
            selmask = lanes < TOPK
            for b in range(B):
                pltpu.sync_copy(scores_hbm.at[b], row_v)
                row = row_v[...]
                w1 = jnp.max(row)
                i1 = jnp.min(jnp.where(row == w1, lanes, 16))
                row2 = jnp.where(lanes == i1, -1.0, row)
                w2 = jnp.max(row2)
                i2 = jnp.min(jnp.where(row2 == w2, lanes, 16))
                wv = jnp.where(lanes == 0, w1,
                               jnp.where(lanes == 1, w2, 0.0)) * temp_v[...]
                iv = jnp.where(lanes == 0, i1, jnp.where(lanes == 1, i2, 0))
                w_v[...] = wv
                i_v[...] = iv
                pltpu.sync_copy(w_v, w_hbm.at[b])
                pltpu.sync_copy(i_v, i_hbm.at[b])
                plsc.addupdate_scatter(usage_v, [iv], wv, mask=selmask)
            u = usage_v[...]
            zv = jnp.zeros((16,), jnp.float32)
            meanv = zv + jnp.sum(u) * (1.0 / E)
            frac = u / (meanv + 1e-6)
            emask = lanes < E
            fr = jnp.where(emask, frac, 0.0)
            muv = zv + jnp.sum(fr) * (1.0 / E)
            d = jnp.where(emask, fr - muv, 0.0)
            varv = zv + jnp.sum(d * d) * (1.0 / (E - 1))
            pltpu.sync_copy(aux_hbm, bal_v)
            bal_v[...] = BALANCE_W * varv + bal_v[...]
            pltpu.sync_copy(bal_v, bal_hbm)

    return k(scores16, aux16, temp16)


def _wprep_kernel(ids_ref, w_ref, wq_ref, *, rows, chunk):
    mn = jnp.float32(0.0)
    mx = jnp.float32(0.0)
    for c in range(rows // chunk):
        blk = w_ref[0, pl.ds(c * chunk, chunk), :]
        mn = jnp.minimum(mn, jnp.min(blk))
        mx = jnp.maximum(mx, jnp.max(blk))
    sw = jnp.maximum((mx - mn) / (QMAX - QMIN), 1e-8)
    zw = jnp.clip(jnp.round(QMIN - mn / sw), QMIN, QMAX)
    for c in range(rows // chunk):
        blk = w_ref[0, pl.ds(c * chunk, chunk), :]
        q = jnp.clip(jnp.round(blk / sw + zw), QMIN, QMAX) - zw
        wq_ref[0, pl.ds(c * chunk, chunk), :] = (q * sw).astype(jnp.bfloat16)


def _wprep(wst, ids, nslots):
    out_f, k = wst.shape[1], wst.shape[2]
    grid_spec = pltpu.PrefetchScalarGridSpec(
        num_scalar_prefetch=1,
        grid=(nslots,),
        in_specs=[pl.BlockSpec((1, out_f, k), lambda s, ids: (ids[s], 0, 0))],
        out_specs=pl.BlockSpec((1, out_f, k), lambda s, ids: (s, 0, 0)),
    )
    return pl.pallas_call(
        functools.partial(_wprep_kernel, rows=out_f, chunk=256),
        grid_spec=grid_spec,
        out_shape=jax.ShapeDtypeStruct((nslots, out_f, k), jnp.bfloat16),
    )(ids, wst)


def _main_kernel(ids_ref, x_ref, wq_ref, b_ref, a_ref, o_ref, mn_ref, mx_ref):
    n = pl.program_id(1)
    m = pl.program_id(2)

    x = x_ref[0]
    p = _dot_t(x, b_ref[0]).astype(jnp.bfloat16)
    acc = _dot_t(x, wq_ref[0])
    acc = acc + _dot_t(p, a_ref[0])
    o_ref[0] = acc

    @pl.when((n == 0) & (m == 0))
    def _():
        mn_ref[...] = jnp.zeros_like(mn_ref)
        mx_ref[...] = jnp.zeros_like(mx_ref)

    mn_ref[...] = jnp.minimum(mn_ref[...], jnp.min(acc))
    mx_ref[...] = jnp.maximum(mx_ref[...], jnp.max(acc))


def _main_mm(x3, wq, bst, ast, ids, nslots, x_per_slot):
    bm, bn = 1024, 2048
    out_f = wq.shape[1]
    k = wq.shape[2]
    if x_per_slot:
        x_imap = lambda s, n, m, ids: (s, m, 0)
    else:
        x_imap = lambda s, n, m, ids: (0, m, 0)
    grid_spec = pltpu.PrefetchScalarGridSpec(
        num_scalar_prefetch=1,
        grid=(nslots, out_f // bn, M // bm),
        in_specs=[
            pl.BlockSpec((1, bm, k), x_imap),
            pl.BlockSpec((1, bn, k), lambda s, n, m, ids: (s, n, 0)),
            pl.BlockSpec((1, RANK, k), lambda s, n, m, ids: (ids[s], 0, 0)),
            pl.BlockSpec((1, bn, RANK), lambda s, n, m, ids: (ids[s], n, 0)),
        ],
        out_specs=[
            pl.BlockSpec((1, bm, bn), lambda s, n, m, ids: (s, m, n)),
            pl.BlockSpec((1, 1, 1), lambda s, n, m, ids: (s, 0, 0)),
            pl.BlockSpec((1, 1, 1), lambda s, n, m, ids: (s, 0, 0)),
        ],
    )
    return pl.pallas_call(
        _main_kernel,
        grid_spec=grid_spec,
        out_shape=[
            jax.ShapeDtypeStruct((nslots, M, out_f), jnp.float32),
            jax.ShapeDtypeStruct((nslots, 1, 1), jnp.float32),
            jax.ShapeDtypeStruct((nslots, 1, 1), jnp.float32),
        ],
    )(ids, x3, wq, bst, ast)


def _main2_kernel(ids_ref, s1_ref, z1_ref, s3_ref, z3_ref, r1_ref, r3_ref,
                  wq_ref, b_ref, a_ref, o_ref, mn_ref, mx_ref):
    s = pl.program_id(0)
    m = pl.program_id(1)
    d1 = _fqa(r1_ref[0], s1_ref[s], z1_ref[s])
    d3 = _fqa(r3_ref[0], s3_ref[s], z3_ref[s])
    x = (d1 * (1.0 / (1.0 + jnp.exp(-d3)))).astype(jnp.bfloat16)
    p = _dot_t(x, b_ref[0]).astype(jnp.bfloat16)
    acc = _dot_t(x, wq_ref[0])
    acc = acc + _dot_t(p, a_ref[0])
    o_ref[0] = acc

    @pl.when(m == 0)
    def _():
        mn_ref[...] = jnp.zeros_like(mn_ref)
        mx_ref[...] = jnp.zeros_like(mx_ref)

    mn_ref[...] = jnp.minimum(mn_ref[...], jnp.min(acc))
    mx_ref[...] = jnp.maximum(mx_ref[...], jnp.max(acc))


def _main2_mm(r1, r3, s1, z1, s3, z3, wq, bst, ast, ids, nslots):
    bm = 512
    out_f = wq.shape[1]
    k = wq.shape[2]
    grid_spec = pltpu.PrefetchScalarGridSpec(
        num_scalar_prefetch=1,
        grid=(nslots, M // bm),
        in_specs=[
            pl.BlockSpec(memory_space=pltpu.SMEM),
            pl.BlockSpec(memory_space=pltpu.SMEM),
            pl.BlockSpec(memory_space=pltpu.SMEM),
            pl.BlockSpec(memory_space=pltpu.SMEM),
            pl.BlockSpec((1, bm, INTER), lambda s, m, ids: (s, m, 0)),
            pl.BlockSpec((1, bm, INTER), lambda s, m, ids: (s, m, 0)),
            pl.BlockSpec((1, out_f, k), lambda s, m, ids: (s, 0, 0)),
            pl.BlockSpec((1, RANK, k), lambda s, m, ids: (ids[s], 0, 0)),
            pl.BlockSpec((1, out_f, RANK), lambda s, m, ids: (ids[s], 0, 0)),
        ],
        out_specs=[
            pl.BlockSpec((1, bm, out_f), lambda s, m, ids: (s, m, 0)),
            pl.BlockSpec((1, 1, 1), lambda s, m, ids: (s, 0, 0)),
            pl.BlockSpec((1, 1, 1), lambda s, m, ids: (s, 0, 0)),
        ],
    )
    return pl.pallas_call(
        _main2_kernel,
        grid_spec=grid_spec,
        out_shape=[
            jax.ShapeDtypeStruct((nslots, M, out_f), jnp.float32),
            jax.ShapeDtypeStruct((nslots, 1, 1), jnp.float32),
            jax.ShapeDtypeStruct((nslots, 1, 1), jnp.float32),
        ],
    )(ids, s1, z1, s3, z3, r1, r3, wq, bst, ast)


def _combine_kernel(g_ref, sa_ref, za_ref, sb_ref, zb_ref,
                    r2e_ref, r2s_ref, y_ref, *, bm):
    m = pl.program_id(0)
    b = (m * bm) // T
    acc = _fqa(_fqa(r2s_ref[...], sa_ref[2 * TOPK], za_ref[2 * TOPK]),
               sb_ref[2 * TOPK], zb_ref[2 * TOPK])
    for s in range(B * TOPK):
        v = _fqa(_fqa(r2e_ref[s], sa_ref[s], za_ref[s]),
                 sb_ref[s], zb_ref[s])
        acc = acc + g_ref[s, b] * v
    y_ref[...] = acc


def _combine(r2e, r2s, gmat, sa, za, sb, zb):
    bm, bn = 512, 512
    return pl.pallas_call(
        functools.partial(_combine_kernel, bm=bm),
        grid=(M // bm, DIM // bn),
        in_specs=[pl.BlockSpec(memory_space=pltpu.SMEM)] * 5
        + [pl.BlockSpec((B * TOPK, bm, bn), lambda m, n: (0, m, n)),
           pl.BlockSpec((bm, bn), lambda m, n: (m, n))],
        out_specs=pl.BlockSpec((bm, bn), lambda m, n: (m, n)),
        out_shape=jax.ShapeDtypeStruct((M, DIM), jnp.float32),
    )(gmat, sa, za, sb, zb, r2e, r2s)


def _scale_zp(mn, mx):
    s = jnp.maximum((mx - mn) / (QMAX - QMIN), 1e-8)
    z = jnp.clip(jnp.round(QMIN - mn / s), QMIN, QMAX)
    return s, z


def _lora_layer(x3, wp, ids, nslots, x_per_slot):
    wst, ast, bst = wp['weight'], wp['lora_A'], wp['lora_B']
    if wst.ndim == 2:
        wst = wst.reshape((1,) + wst.shape)
        ast = ast.reshape((1,) + ast.shape)
        bst = bst.reshape((1,) + bst.shape)
    wq = _wprep(wst, ids, nslots)
    r, mn, mx = _main_mm(x3, wq, bst, ast, ids, nslots, x_per_slot)
    return r, mn[:, 0, 0], mx[:, 0, 0]


def _expert_stack(x3, prm, ids, nslots, x_per_slot):
    r3, mn3, mx3 = _lora_layer(x3, prm['w3'], ids, nslots, x_per_slot)
    r1, mn1, mx1 = _lora_layer(x3, prm['w1'], ids, nslots, x_per_slot)
    s1, z1 = _scale_zp(mn1, mx1)
    s3, z3 = _scale_zp(mn3, mx3)
    wp = prm['w2']
    wst, ast, bst = wp['weight'], wp['lora_A'], wp['lora_B']
    if wst.ndim == 2:
        wst = wst.reshape((1,) + wst.shape)
        ast = ast.reshape((1,) + ast.shape)
        bst = bst.reshape((1,) + bst.shape)
    wq2 = _wprep(wst, ids, nslots)
    r2, mn2, mx2 = _main2_mm(r1, r3, s1, z1, s3, z3, wq2, bst, ast, ids, nslots)
    return r2, mn2[:, 0, 0], mx2[:, 0, 0]


def kernel(x, params):
    X3 = x.reshape(1, M, DIM).astype(jnp.bfloat16)
    xm = _mean(x)
    scores, aux = _router(xm, params['router'])

    scores16 = jnp.concatenate(
        [scores, jnp.full((B, 16 - E), -1.0, jnp.float32)], axis=1)
    aux16 = jnp.full((16,), aux[0, 0], jnp.float32)
    temp16 = jnp.full((16,), jnp.maximum(params['router']['temp'], 0.1),
                      jnp.float32)
    w16, i16, bal16 = _sc_route(scores16, aux16, temp16)
    wts = w16[:, :TOPK]
    idx = i16[:, :TOPK]

    ids = idx.reshape(-1).astype(jnp.int32)
    ids0 = jnp.zeros((1,), jnp.int32)

    r2s, mn2s, mx2s = _expert_stack(X3, params['shared'], ids0, 1, False)
    r2e, mn2e, mx2e = _expert_stack(X3, params['experts'], ids, B * TOPK, False)

    mn2 = jnp.concatenate([mn2e, mn2s])
    mx2 = jnp.concatenate([mx2e, mx2s])
    sa, za = _scale_zp(mn2, mx2)
    dmn = _fqa(mn2, sa, za)
    dmx = _fqa(mx2, sa, za)
    sb, zb = _scale_zp(dmn, dmx)

    gvec = wts.reshape(-1)
    sb_of_slot = jnp.arange(B * TOPK) // TOPK
    gmat = gvec[:, None] * (sb_of_slot[:, None] ==
                            jnp.arange(B)[None, :]).astype(jnp.float32)

    y = _combine(r2e, r2s[0], gmat, sa, za, sb, zb)
    return y.reshape(B, T, DIM), bal16[0]


# speedup vs baseline: 1.2108x; 1.0568x over previous
import functools

import jax
import jax.numpy as jnp
from jax.experimental import pallas as pl
from jax.experimental.pallas import tpu as pltpu
from jax.experimental.pallas import tpu_sc as plsc

DIM = 2048
INTER = 2048
E = 8
TOPK = 2
RANK = 128
B = 2
T = 2048
M = B * T
QMIN, QMAX = -128.0, 127.0
BALANCE_W, ENTROPY_W, ZLOSS_W = 0.3, 0.1, 0.0001

def _dot_t(a, b):
    return jax.lax.dot_general(a.astype(jnp.bfloat16), b.astype(jnp.bfloat16),
                               (((1,), (1,)), ((), ())),
                               preferred_element_type=jnp.float32)


def _fqa(v, s, z):
    q = jnp.clip(jnp.round(v / s + z), QMIN, QMAX)
    return (q - z) * s


def _mean_kernel(x_ref, o_ref):
    @pl.when(pl.program_id(0) == 0)
    def _():
        o_ref[...] = jnp.zeros_like(o_ref)

    o_ref[...] += jnp.sum(x_ref[...], axis=1) * (1.0 / T)


def _mean(x):
    bt = 256
    return pl.pallas_call(
        _mean_kernel,
        grid=(T // bt,),
        in_specs=[pl.BlockSpec((B, bt, DIM), lambda t: (0, t, 0))],
        out_specs=pl.BlockSpec((B, DIM), lambda t: (0, 0)),
        out_shape=jax.ShapeDtypeStruct((B, DIM), jnp.float32),
    )(x)


def _router_kernel(xm_ref, wp1_ref, bp1_ref, wp2_ref, bp2_ref, ws_ref, bs_ref,
                   temp_ref, s_out, l_out):
    xm = xm_ref[...]
    h = jnp.maximum(_dot_t(xm, wp1_ref[...]) + bp1_ref[...], 0.0)
    pol_l = _dot_t(h, wp2_ref[...]) + bp2_ref[...]
    pol_m = jnp.max(pol_l, axis=-1, keepdims=True)
    pol_e = jnp.exp(pol_l - pol_m)
    policy = pol_e / jnp.sum(pol_e, axis=-1, keepdims=True)

    temp = jnp.maximum(temp_ref[0], 0.1)
    base = (_dot_t(xm, ws_ref[...]) + bs_ref[...]) / temp

    sc_l = (base + policy) * 0.5
    sc_m = jnp.max(sc_l, axis=-1, keepdims=True)
    sc_e = jnp.exp(sc_l - sc_m)
    scores = sc_e / jnp.sum(sc_e, axis=-1, keepdims=True)

    entropy = -jnp.sum(scores * jnp.log(scores + 1e-6), axis=-1, keepdims=True)
    entropy_loss = -ENTROPY_W * jnp.mean(entropy)

    b_m = jnp.max(base, axis=-1, keepdims=True)
    lse = jnp.log(jnp.sum(jnp.exp(base - b_m), axis=-1, keepdims=True)) + b_m
    z_loss = ZLOSS_W * jnp.mean(lse * lse)

    s_out[...] = scores
    l_out[...] = jnp.full((1, 1), entropy_loss + z_loss, jnp.float32)


def _router(xm, rp):
    return pl.pallas_call(
        _router_kernel,
        in_specs=[pl.BlockSpec(memory_space=pltpu.VMEM)] * 7
        + [pl.BlockSpec(memory_space=pltpu.SMEM)],
        out_specs=[pl.BlockSpec(memory_space=pltpu.VMEM)] * 2,
        out_shape=[
            jax.ShapeDtypeStruct((B, E), jnp.float32),
            jax.ShapeDtypeStruct((1, 1), jnp.float32),
        ],
    )(xm, rp['Wp1'], rp['bp1'].reshape(1, 256), rp['Wp2'],
      rp['bp2'].reshape(1, E), rp['Ws'], rp['bs'].reshape(1, E),
      rp['temp'].reshape(1))


def _sc_route(scores16, aux16, temp16):
    @functools.partial(
        pl.kernel,
        out_type=[
            jax.ShapeDtypeStruct((B, 16), jnp.float32),
            jax.ShapeDtypeStruct((B, 16), jnp.int32),
            jax.ShapeDtypeStruct((16,), jnp.float32),
        ],
        mesh=plsc.VectorSubcoreMesh(core_axis_name="c", subcore_axis_name="s"),
        compiler_params=pltpu.CompilerParams(needs_layout_passes=False),
        scratch_types=[
            pltpu.VMEM((16,), jnp.float32),
            pltpu.VMEM((16,), jnp.float32),
            pltpu.VMEM((16,), jnp.int32),
            pltpu.VMEM((16,), jnp.float32),
            pltpu.VMEM((16,), jnp.float32),
            pltpu.VMEM((16,), jnp.float32),
        ],
    )
    def k(scores_hbm, aux_hbm, temp_hbm, w_hbm, i_hbm, bal_hbm,
          row_v, w_v, i_v, usage_v, bal_v, temp_v):
        @pl.when((jax.lax.axis_index("c") == 0)
                 & (jax.lax.axis_index("s") == 0))
        def _():
            usage_v[...] = jnp.zeros((16,), jnp.float32)
            pltpu.sync_copy(temp_hbm, temp_v)
            lanes = jax.lax.iota(jnp.int32, 16)
            selmask = lanes < TOPK
            for b in range(B):
                pltpu.sync_copy(scores_hbm.at[b], row_v)
                row = row_v[...]
                w1 = jnp.max(row)
                i1 = jnp.min(jnp.where(row == w1, lanes, 16))
                row2 = jnp.where(lanes == i1, -1.0, row)
                w2 = jnp.max(row2)
                i2 = jnp.min(jnp.where(row2 == w2, lanes, 16))
                wv = jnp.where(lanes == 0, w1,
                               jnp.where(lanes == 1, w2, 0.0)) * temp_v[...]
                iv = jnp.where(lanes == 0, i1, jnp.where(lanes == 1, i2, 0))
                w_v[...] = wv
                i_v[...] = iv
                pltpu.sync_copy(w_v, w_hbm.at[b])
                pltpu.sync_copy(i_v, i_hbm.at[b])
                plsc.addupdate_scatter(usage_v, [iv], wv, mask=selmask)
            u = usage_v[...]
            zv = jnp.zeros((16,), jnp.float32)
            meanv = zv + jnp.sum(u) * (1.0 / E)
            frac = u / (meanv + 1e-6)
            emask = lanes < E
            fr = jnp.where(emask, frac, 0.0)
            muv = zv + jnp.sum(fr) * (1.0 / E)
            d = jnp.where(emask, fr - muv, 0.0)
            varv = zv + jnp.sum(d * d) * (1.0 / (E - 1))
            pltpu.sync_copy(aux_hbm, bal_v)
            bal_v[...] = BALANCE_W * varv + bal_v[...]
            pltpu.sync_copy(bal_v, bal_hbm)

    return k(scores16, aux16, temp16)


def _wprep_kernel(ids_ref, w_ref, wq_ref, *, rows, chunk):
    mn = jnp.float32(0.0)
    mx = jnp.float32(0.0)
    for c in range(rows // chunk):
        blk = w_ref[0, pl.ds(c * chunk, chunk), :]
        mn = jnp.minimum(mn, jnp.min(blk))
        mx = jnp.maximum(mx, jnp.max(blk))
    sw = jnp.maximum((mx - mn) / (QMAX - QMIN), 1e-8)
    zw = jnp.clip(jnp.round(QMIN - mn / sw), QMIN, QMAX)
    for c in range(rows // chunk):
        blk = w_ref[0, pl.ds(c * chunk, chunk), :]
        q = jnp.clip(jnp.round(blk / sw + zw), QMIN, QMAX) - zw
        wq_ref[0, pl.ds(c * chunk, chunk), :] = (q * sw).astype(jnp.bfloat16)


def _wprep(wst, ids, nslots):
    out_f, k = wst.shape[1], wst.shape[2]
    grid_spec = pltpu.PrefetchScalarGridSpec(
        num_scalar_prefetch=1,
        grid=(nslots,),
        in_specs=[pl.BlockSpec((1, out_f, k), lambda s, ids: (ids[s], 0, 0))],
        out_specs=pl.BlockSpec((1, out_f, k), lambda s, ids: (s, 0, 0)),
    )
    return pl.pallas_call(
        functools.partial(_wprep_kernel, rows=out_f, chunk=256),
        grid_spec=grid_spec,
        out_shape=jax.ShapeDtypeStruct((nslots, out_f, k), jnp.bfloat16),
    )(ids, wst)


def _main_kernel(ids_ref, x_ref, wq_ref, b_ref, a_ref, o_ref, mn_ref, mx_ref):
    n = pl.program_id(1)
    m = pl.program_id(2)

    x = x_ref[0]
    p = _dot_t(x, b_ref[0]).astype(jnp.bfloat16)
    acc = _dot_t(x, wq_ref[0])
    acc = acc + _dot_t(p, a_ref[0])
    o_ref[0] = acc

    @pl.when((n == 0) & (m == 0))
    def _():
        mn_ref[...] = jnp.zeros_like(mn_ref)
        mx_ref[...] = jnp.zeros_like(mx_ref)

    mn_ref[...] = jnp.minimum(mn_ref[...], jnp.min(acc))
    mx_ref[...] = jnp.maximum(mx_ref[...], jnp.max(acc))


def _main_mm(x3, wq, bst, ast, ids, nslots, x_per_slot):
    bm, bn = 1024, 2048
    out_f = wq.shape[1]
    k = wq.shape[2]
    if x_per_slot:
        x_imap = lambda s, n, m, ids: (s, m, 0)
    else:
        x_imap = lambda s, n, m, ids: (0, m, 0)
    grid_spec = pltpu.PrefetchScalarGridSpec(
        num_scalar_prefetch=1,
        grid=(nslots, out_f // bn, M // bm),
        in_specs=[
            pl.BlockSpec((1, bm, k), x_imap),
            pl.BlockSpec((1, bn, k), lambda s, n, m, ids: (s, n, 0)),
            pl.BlockSpec((1, RANK, k), lambda s, n, m, ids: (ids[s], 0, 0)),
            pl.BlockSpec((1, bn, RANK), lambda s, n, m, ids: (ids[s], n, 0)),
        ],
        out_specs=[
            pl.BlockSpec((1, bm, bn), lambda s, n, m, ids: (s, m, n)),
            pl.BlockSpec((1, 1, 1), lambda s, n, m, ids: (s, 0, 0)),
            pl.BlockSpec((1, 1, 1), lambda s, n, m, ids: (s, 0, 0)),
        ],
    )
    return pl.pallas_call(
        _main_kernel,
        grid_spec=grid_spec,
        out_shape=[
            jax.ShapeDtypeStruct((nslots, M, out_f), jnp.float32),
            jax.ShapeDtypeStruct((nslots, 1, 1), jnp.float32),
            jax.ShapeDtypeStruct((nslots, 1, 1), jnp.float32),
        ],
    )(ids, x3, wq, bst, ast)


def _main13_kernel(ids_ref, x_ref, wq3_ref, b3_ref, a3_ref,
                   wq1_ref, b1_ref, a1_ref,
                   o3_ref, o1_ref, mn3_ref, mx3_ref, mn1_ref, mx1_ref):
    m = pl.program_id(1)
    x = x_ref[0]

    @pl.when(m == 0)
    def _():
        mn3_ref[...] = jnp.zeros_like(mn3_ref)
        mx3_ref[...] = jnp.zeros_like(mx3_ref)
        mn1_ref[...] = jnp.zeros_like(mn1_ref)
        mx1_ref[...] = jnp.zeros_like(mx1_ref)

    p3 = _dot_t(x, b3_ref[0]).astype(jnp.bfloat16)
    acc3 = _dot_t(x, wq3_ref[0]) + _dot_t(p3, a3_ref[0])
    o3_ref[0] = acc3
    mn3_ref[...] = jnp.minimum(mn3_ref[...], jnp.min(acc3))
    mx3_ref[...] = jnp.maximum(mx3_ref[...], jnp.max(acc3))

    p1 = _dot_t(x, b1_ref[0]).astype(jnp.bfloat16)
    acc1 = _dot_t(x, wq1_ref[0]) + _dot_t(p1, a1_ref[0])
    o1_ref[0] = acc1
    mn1_ref[...] = jnp.minimum(mn1_ref[...], jnp.min(acc1))
    mx1_ref[...] = jnp.maximum(mx1_ref[...], jnp.max(acc1))


def _main13_mm(x3, wq3, w3p, wq1, w1p, ids, nslots):
    bm = 256
    grid_spec = pltpu.PrefetchScalarGridSpec(
        num_scalar_prefetch=1,
        grid=(nslots, M // bm),
        in_specs=[
            pl.BlockSpec((1, bm, DIM), lambda s, m, ids: (0, m, 0)),
            pl.BlockSpec((1, INTER, DIM), lambda s, m, ids: (s, 0, 0)),
            pl.BlockSpec((1, RANK, DIM), lambda s, m, ids: (ids[s], 0, 0)),
            pl.BlockSpec((1, INTER, RANK), lambda s, m, ids: (ids[s], 0, 0)),
            pl.BlockSpec((1, INTER, DIM), lambda s, m, ids: (s, 0, 0)),
            pl.BlockSpec((1, RANK, DIM), lambda s, m, ids: (ids[s], 0, 0)),
            pl.BlockSpec((1, INTER, RANK), lambda s, m, ids: (ids[s], 0, 0)),
        ],
        out_specs=[
            pl.BlockSpec((1, bm, INTER), lambda s, m, ids: (s, m, 0)),
            pl.BlockSpec((1, bm, INTER), lambda s, m, ids: (s, m, 0)),
            pl.BlockSpec((1, 1, 1), lambda s, m, ids: (s, 0, 0)),
            pl.BlockSpec((1, 1, 1), lambda s, m, ids: (s, 0, 0)),
            pl.BlockSpec((1, 1, 1), lambda s, m, ids: (s, 0, 0)),
            pl.BlockSpec((1, 1, 1), lambda s, m, ids: (s, 0, 0)),
        ],
    )
    return pl.pallas_call(
        _main13_kernel,
        grid_spec=grid_spec,
        out_shape=[
            jax.ShapeDtypeStruct((nslots, M, INTER), jnp.float32),
            jax.ShapeDtypeStruct((nslots, M, INTER), jnp.float32),
            jax.ShapeDtypeStruct((nslots, 1, 1), jnp.float32),
            jax.ShapeDtypeStruct((nslots, 1, 1), jnp.float32),
            jax.ShapeDtypeStruct((nslots, 1, 1), jnp.float32),
            jax.ShapeDtypeStruct((nslots, 1, 1), jnp.float32),
        ],
    )(ids, x3, wq3, w3p['lora_B'], w3p['lora_A'], wq1, w1p['lora_B'],
      w1p['lora_A'])


def _main2_kernel(ids_ref, s1_ref, z1_ref, s3_ref, z3_ref, r1_ref, r3_ref,
                  wq_ref, b_ref, a_ref, o_ref, mn_ref, mx_ref):
    s = pl.program_id(0)
    m = pl.program_id(1)
    d1 = _fqa(r1_ref[0], s1_ref[s], z1_ref[s])
    d3 = _fqa(r3_ref[0], s3_ref[s], z3_ref[s])
    x = (d1 * (1.0 / (1.0 + jnp.exp(-d3)))).astype(jnp.bfloat16)
    p = _dot_t(x, b_ref[0]).astype(jnp.bfloat16)
    acc = _dot_t(x, wq_ref[0])
    acc = acc + _dot_t(p, a_ref[0])
    o_ref[0] = acc

    @pl.when(m == 0)
    def _():
        mn_ref[...] = jnp.zeros_like(mn_ref)
        mx_ref[...] = jnp.zeros_like(mx_ref)

    mn_ref[...] = jnp.minimum(mn_ref[...], jnp.min(acc))
    mx_ref[...] = jnp.maximum(mx_ref[...], jnp.max(acc))


def _main2_mm(r1, r3, s1, z1, s3, z3, wq, bst, ast, ids, nslots):
    bm = 512
    out_f = wq.shape[1]
    k = wq.shape[2]
    grid_spec = pltpu.PrefetchScalarGridSpec(
        num_scalar_prefetch=1,
        grid=(nslots, M // bm),
        in_specs=[
            pl.BlockSpec(memory_space=pltpu.SMEM),
            pl.BlockSpec(memory_space=pltpu.SMEM),
            pl.BlockSpec(memory_space=pltpu.SMEM),
            pl.BlockSpec(memory_space=pltpu.SMEM),
            pl.BlockSpec((1, bm, INTER), lambda s, m, ids: (s, m, 0)),
            pl.BlockSpec((1, bm, INTER), lambda s, m, ids: (s, m, 0)),
            pl.BlockSpec((1, out_f, k), lambda s, m, ids: (s, 0, 0)),
            pl.BlockSpec((1, RANK, k), lambda s, m, ids: (ids[s], 0, 0)),
            pl.BlockSpec((1, out_f, RANK), lambda s, m, ids: (ids[s], 0, 0)),
        ],
        out_specs=[
            pl.BlockSpec((1, bm, out_f), lambda s, m, ids: (s, m, 0)),
            pl.BlockSpec((1, 1, 1), lambda s, m, ids: (s, 0, 0)),
            pl.BlockSpec((1, 1, 1), lambda s, m, ids: (s, 0, 0)),
        ],
    )
    return pl.pallas_call(
        _main2_kernel,
        grid_spec=grid_spec,
        out_shape=[
            jax.ShapeDtypeStruct((nslots, M, out_f), jnp.float32),
            jax.ShapeDtypeStruct((nslots, 1, 1), jnp.float32),
            jax.ShapeDtypeStruct((nslots, 1, 1), jnp.float32),
        ],
    )(ids, s1, z1, s3, z3, r1, r3, wq, bst, ast)


def _combine_kernel(g_ref, sa_ref, za_ref, sb_ref, zb_ref,
                    r2e_ref, r2s_ref, y_ref, *, bm):
    m = pl.program_id(0)
    b = (m * bm) // T
    acc = _fqa(_fqa(r2s_ref[...], sa_ref[2 * TOPK], za_ref[2 * TOPK]),
               sb_ref[2 * TOPK], zb_ref[2 * TOPK])
    for s in range(B * TOPK):
        v = _fqa(_fqa(r2e_ref[s], sa_ref[s], za_ref[s]),
                 sb_ref[s], zb_ref[s])
        acc = acc + g_ref[s, b] * v
    y_ref[...] = acc


def _combine(r2e, r2s, gmat, sa, za, sb, zb):
    bm, bn = 512, 512
    return pl.pallas_call(
        functools.partial(_combine_kernel, bm=bm),
        grid=(M // bm, DIM // bn),
        in_specs=[pl.BlockSpec(memory_space=pltpu.SMEM)] * 5
        + [pl.BlockSpec((B * TOPK, bm, bn), lambda m, n: (0, m, n)),
           pl.BlockSpec((bm, bn), lambda m, n: (m, n))],
        out_specs=pl.BlockSpec((bm, bn), lambda m, n: (m, n)),
        out_shape=jax.ShapeDtypeStruct((M, DIM), jnp.float32),
    )(gmat, sa, za, sb, zb, r2e, r2s)


def _scale_zp(mn, mx):
    s = jnp.maximum((mx - mn) / (QMAX - QMIN), 1e-8)
    z = jnp.clip(jnp.round(QMIN - mn / s), QMIN, QMAX)
    return s, z


def _lora_layer(x3, wp, ids, nslots, x_per_slot):
    wst, ast, bst = wp['weight'], wp['lora_A'], wp['lora_B']
    if wst.ndim == 2:
        wst = wst.reshape((1,) + wst.shape)
        ast = ast.reshape((1,) + ast.shape)
        bst = bst.reshape((1,) + bst.shape)
    wq = _wprep(wst, ids, nslots)
    r, mn, mx = _main_mm(x3, wq, bst, ast, ids, nslots, x_per_slot)
    return r, mn[:, 0, 0], mx[:, 0, 0]


def _norm_wp(wp):
    if wp['weight'].ndim == 2:
        return jax.tree_util.tree_map(lambda a: a.reshape((1,) + a.shape), wp)
    return wp


def _expert_stack(x3, prm, ids, nslots, x_per_slot):
    w3p = _norm_wp(prm['w3'])
    w1p = _norm_wp(prm['w1'])
    wq3 = _wprep(w3p['weight'], ids, nslots)
    wq1 = _wprep(w1p['weight'], ids, nslots)
    r3, r1, mn3, mx3, mn1, mx1 = _main13_mm(x3, wq3, w3p, wq1, w1p, ids,
                                            nslots)
    mn3, mx3 = mn3[:, 0, 0], mx3[:, 0, 0]
    mn1, mx1 = mn1[:, 0, 0], mx1[:, 0, 0]
    s1, z1 = _scale_zp(mn1, mx1)
    s3, z3 = _scale_zp(mn3, mx3)
    wp = prm['w2']
    wst, ast, bst = wp['weight'], wp['lora_A'], wp['lora_B']
    if wst.ndim == 2:
        wst = wst.reshape((1,) + wst.shape)
        ast = ast.reshape((1,) + ast.shape)
        bst = bst.reshape((1,) + bst.shape)
    wq2 = _wprep(wst, ids, nslots)
    r2, mn2, mx2 = _main2_mm(r1, r3, s1, z1, s3, z3, wq2, bst, ast, ids, nslots)
    return r2, mn2[:, 0, 0], mx2[:, 0, 0]


def kernel(x, params):
    X3 = x.reshape(1, M, DIM).astype(jnp.bfloat16)
    xm = _mean(x)
    scores, aux = _router(xm, params['router'])

    scores16 = jnp.concatenate(
        [scores, jnp.full((B, 16 - E), -1.0, jnp.float32)], axis=1)
    aux16 = jnp.full((16,), aux[0, 0], jnp.float32)
    temp16 = jnp.full((16,), jnp.maximum(params['router']['temp'], 0.1),
                      jnp.float32)
    w16, i16, bal16 = _sc_route(scores16, aux16, temp16)
    wts = w16[:, :TOPK]
    idx = i16[:, :TOPK]

    ids = idx.reshape(-1).astype(jnp.int32)
    ids0 = jnp.zeros((1,), jnp.int32)

    r2s, mn2s, mx2s = _expert_stack(X3, params['shared'], ids0, 1, False)
    r2e, mn2e, mx2e = _expert_stack(X3, params['experts'], ids, B * TOPK, False)

    mn2 = jnp.concatenate([mn2e, mn2s])
    mx2 = jnp.concatenate([mx2e, mx2s])
    sa, za = _scale_zp(mn2, mx2)
    dmn = _fqa(mn2, sa, za)
    dmx = _fqa(mx2, sa, za)
    sb, zb = _scale_zp(dmn, dmx)

    gvec = wts.reshape(-1)
    sb_of_slot = jnp.arange(B * TOPK) // TOPK
    gmat = gvec[:, None] * (sb_of_slot[:, None] ==
                            jnp.arange(B)[None, :]).astype(jnp.float32)

    y = _combine(r2e, r2s[0], gmat, sa, za, sb, zb)
    return y.reshape(B, T, DIM), bal16[0]
